# baseline (speedup 1.0000x reference)
import functools

import jax
import jax.numpy as jnp
from jax import lax
from jax.experimental import pallas as pl
from jax.experimental.pallas import tpu as pltpu
from jax.experimental.pallas import tpu_sc as plsc

VOCAB = 100000
D_MODEL = 1024
MAX_SEQ = 2048
BATCH = 4

N_TOK = BATCH * MAX_SEQ
NUM_CORES = 2
NUM_SUBCORES = 16
NW = NUM_CORES * NUM_SUBCORES
BPW = N_TOK // NW
CHUNK = 32
NCHUNK = BPW // CHUNK
LANES = 16


def _positional_encoding():
    pos = jnp.arange(MAX_SEQ, dtype=jnp.float32)[:, None]
    i = jnp.arange(0, D_MODEL, 2, dtype=jnp.float32)
    denom = jnp.power(10000.0, i / D_MODEL)
    even = jnp.sin(pos / denom)
    odd = jnp.cos(pos / denom)
    pe = jnp.zeros((MAX_SEQ, D_MODEL), dtype=jnp.float32)
    pe = pe.at[:, 0::2].set(even)
    pe = pe.at[:, 1::2].set(odd)
    return pe


def _sc_body(table_hbm, idx_hbm, pe_hbm, out_hbm, idx_v, rows_v, pe_v, sem):
    cid = lax.axis_index("c")
    sid = lax.axis_index("s")
    wid = sid * NUM_CORES + cid
    base = wid * BPW
    pbase = lax.rem(base, MAX_SEQ)

    pltpu.sync_copy(idx_hbm.at[wid], idx_v)

    for j in range(NCHUNK):
        pltpu.async_copy(table_hbm.at[idx_v.at[j]], rows_v, sem).wait()
        pltpu.sync_copy(pe_hbm.at[pl.ds(pbase + j * CHUNK, CHUNK)], pe_v)

        def add_row(r, carry):
            for k in range(D_MODEL // LANES):
                sl = pl.ds(k * LANES, LANES)
                rows_v[r, sl] = rows_v[r, sl] + pe_v[r, sl]
            return carry

        lax.fori_loop(0, CHUNK, add_row, 0)
        pltpu.sync_copy(rows_v, out_hbm.at[pl.ds(base + j * CHUNK, CHUNK)])


@jax.jit
def _embed(x, table):
    pe = _positional_encoding()
    idx = x.reshape(NW, NCHUNK, CHUNK).astype(jnp.int32)
    mesh = plsc.VectorSubcoreMesh(core_axis_name="c", subcore_axis_name="s")
    gather = functools.partial(
        pl.kernel,
        mesh=mesh,
        out_type=jax.ShapeDtypeStruct((N_TOK, D_MODEL), jnp.float32),
        scratch_types=[
            pltpu.VMEM((NCHUNK, CHUNK), jnp.int32),
            pltpu.VMEM((CHUNK, D_MODEL), jnp.float32),
            pltpu.VMEM((CHUNK, D_MODEL), jnp.float32),
            pltpu.SemaphoreType.DMA,
        ],
    )(_sc_body)
    out = gather(table, idx, pe)
    return out.reshape(BATCH, MAX_SEQ, D_MODEL)


def kernel(x, table):
    return _embed(x, table)


# speedup vs baseline: 1.1055x; 1.1055x over previous
import functools

import jax
import jax.numpy as jnp
from jax import lax
from jax.experimental import pallas as pl
from jax.experimental.pallas import tpu as pltpu
from jax.experimental.pallas import tpu_sc as plsc

VOCAB = 100000
D_MODEL = 1024
MAX_SEQ = 2048
BATCH = 4

N_TOK = BATCH * MAX_SEQ
NUM_CORES = 2
NUM_SUBCORES = 16
NW = NUM_CORES * NUM_SUBCORES
BPW = N_TOK // NW
CHUNK = 16
NCHUNK = BPW // CHUNK
LANES = 16


def _positional_encoding():
    pos = jnp.arange(MAX_SEQ, dtype=jnp.float32)[:, None]
    i = jnp.arange(0, D_MODEL, 2, dtype=jnp.float32)
    denom = jnp.power(10000.0, i / D_MODEL)
    even = jnp.sin(pos / denom)
    odd = jnp.cos(pos / denom)
    pe = jnp.zeros((MAX_SEQ, D_MODEL), dtype=jnp.float32)
    pe = pe.at[:, 0::2].set(even)
    pe = pe.at[:, 1::2].set(odd)
    return pe


def _sc_body(table_hbm, idx_hbm, pe_hbm, out_hbm,
             idx_v, rows_v, pe_v, gsem0, gsem1, psem0, psem1, wsem0, wsem1):
    cid = lax.axis_index("c")
    sid = lax.axis_index("s")
    wid = sid * NUM_CORES + cid
    base = wid * BPW
    pbase = lax.rem(base, MAX_SEQ)

    gsem = (gsem0, gsem1)
    psem = (psem0, psem1)
    wsem = (wsem0, wsem1)

    pltpu.sync_copy(idx_hbm.at[wid], idx_v)

    def start(j):
        p = j % 2
        g = pltpu.async_copy(table_hbm.at[idx_v.at[j]], rows_v.at[p], gsem[p])
        pc = pltpu.async_copy(
            pe_hbm.at[pl.ds(pbase + j * CHUNK, CHUNK)], pe_v.at[p], psem[p])
        return g, pc

    inflight = [None] * 2
    wb = [None] * 2

    inflight[0] = start(0)
    for j in range(NCHUNK):
        p = j % 2
        if j + 1 < NCHUNK:
            q = (j + 1) % 2
            if wb[q] is not None:
                wb[q].wait()
                wb[q] = None
            inflight[q] = start(j + 1)
        g, pc = inflight[p]
        g.wait()
        pc.wait()

        def add_row(r, carry):
            for k in range(D_MODEL // LANES):
                sl = pl.ds(k * LANES, LANES)
                rows_v[p, r, sl] = rows_v[p, r, sl] + pe_v[p, r, sl]
            return carry

        lax.fori_loop(0, CHUNK, add_row, 0)
        wb[p] = pltpu.async_copy(
            rows_v.at[p], out_hbm.at[pl.ds(base + j * CHUNK, CHUNK)], wsem[p])
    for w in wb:
        if w is not None:
            w.wait()


@jax.jit
def _embed(x, table):
    pe = _positional_encoding()
    idx = x.reshape(NW, NCHUNK, CHUNK).astype(jnp.int32)
    mesh = plsc.VectorSubcoreMesh(core_axis_name="c", subcore_axis_name="s")
    gather = functools.partial(
        pl.kernel,
        mesh=mesh,
        out_type=jax.ShapeDtypeStruct((N_TOK, D_MODEL), jnp.float32),
        scratch_types=[
            pltpu.VMEM((NCHUNK, CHUNK), jnp.int32),
            pltpu.VMEM((2, CHUNK, D_MODEL), jnp.float32),
            pltpu.VMEM((2, CHUNK, D_MODEL), jnp.float32),
            pltpu.SemaphoreType.DMA,
            pltpu.SemaphoreType.DMA,
            pltpu.SemaphoreType.DMA,
            pltpu.SemaphoreType.DMA,
            pltpu.SemaphoreType.DMA,
            pltpu.SemaphoreType.DMA,
        ],
    )(_sc_body)
    out = gather(table, idx, pe)
    return out.reshape(BATCH, MAX_SEQ, D_MODEL)


def kernel(x, table):
    return _embed(x, table)


# speedup vs baseline: 1.6617x; 1.5032x over previous
import functools

import jax
import jax.numpy as jnp
import numpy as np
from jax import lax
from jax.experimental import pallas as pl
from jax.experimental.pallas import tpu as pltpu
from jax.experimental.pallas import tpu_sc as plsc

VOCAB = 100000
D_MODEL = 1024
MAX_SEQ = 2048
BATCH = 4

N_TOK = BATCH * MAX_SEQ
NUM_CORES = 2
NUM_SUBCORES = 16
NW = NUM_CORES * NUM_SUBCORES
BPW = N_TOK // NW
CHUNK = 16
NCHUNK = BPW // CHUNK
LANES = 16


def _positional_encoding():
    pos = np.arange(MAX_SEQ, dtype=np.float32)[:, None]
    i = np.arange(0, D_MODEL, 2, dtype=np.float32)
    denom = np.power(np.float32(10000.0), i / np.float32(D_MODEL))
    pe = np.zeros((MAX_SEQ, D_MODEL), dtype=np.float32)
    pe[:, 0::2] = np.sin(pos / denom)
    pe[:, 1::2] = np.cos(pos / denom)
    return pe


_PE = _positional_encoding()


def _sc_body(table_hbm, idx_hbm, pe_hbm, out_hbm,
             idx_v, rows_v, pe_v, gsem0, gsem1, psem0, psem1, wsem0, wsem1):
    cid = lax.axis_index("c")
    sid = lax.axis_index("s")
    wid = sid * NUM_CORES + cid
    base = wid * BPW
    pbase = lax.rem(base, MAX_SEQ)

    gsem = (gsem0, gsem1)
    psem = (psem0, psem1)
    wsem = (wsem0, wsem1)

    pltpu.sync_copy(idx_hbm.at[wid], idx_v)

    def start(j):
        p = j % 2
        g = pltpu.async_copy(table_hbm.at[idx_v.at[j]], rows_v.at[p], gsem[p])
        pc = pltpu.async_copy(
            pe_hbm.at[pl.ds(pbase + j * CHUNK, CHUNK)], pe_v.at[p], psem[p])
        return g, pc

    inflight = [None] * 2
    wb = [None] * 2

    inflight[0] = start(0)
    for j in range(NCHUNK):
        p = j % 2
        if j + 1 < NCHUNK:
            q = (j + 1) % 2
            if wb[q] is not None:
                wb[q].wait()
                wb[q] = None
            inflight[q] = start(j + 1)
        g, pc = inflight[p]
        g.wait()
        pc.wait()

        def add_row(r, carry):
            for k in range(D_MODEL // LANES):
                sl = pl.ds(k * LANES, LANES)
                rows_v[p, r, sl] = rows_v[p, r, sl] + pe_v[p, r, sl]
            return carry

        lax.fori_loop(0, CHUNK, add_row, 0)
        wb[p] = pltpu.async_copy(
            rows_v.at[p], out_hbm.at[pl.ds(base + j * CHUNK, CHUNK)], wsem[p])
    for w in wb:
        if w is not None:
            w.wait()


@jax.jit
def _embed(x, table):
    pe = jnp.asarray(_PE)
    idx = x.reshape(NW, NCHUNK, CHUNK).astype(jnp.int32)
    mesh = plsc.VectorSubcoreMesh(core_axis_name="c", subcore_axis_name="s")
    gather = functools.partial(
        pl.kernel,
        mesh=mesh,
        out_type=jax.ShapeDtypeStruct((N_TOK, D_MODEL), jnp.float32),
        scratch_types=[
            pltpu.VMEM((NCHUNK, CHUNK), jnp.int32),
            pltpu.VMEM((2, CHUNK, D_MODEL), jnp.float32),
            pltpu.VMEM((2, CHUNK, D_MODEL), jnp.float32),
            pltpu.SemaphoreType.DMA,
            pltpu.SemaphoreType.DMA,
            pltpu.SemaphoreType.DMA,
            pltpu.SemaphoreType.DMA,
            pltpu.SemaphoreType.DMA,
            pltpu.SemaphoreType.DMA,
        ],
    )(_sc_body)
    out = gather(table, idx, pe)
    return out.reshape(BATCH, MAX_SEQ, D_MODEL)


def kernel(x, table):
    return _embed(x, table)
